# baseline (speedup 1.0000x reference)
import functools

import jax
import jax.numpy as jnp
from jax import lax
from jax.experimental import pallas as pl
from jax.experimental.pallas import tpu as pltpu
from jax.experimental.pallas import tpu_sc as plsc

_ROWS, _COLS = 64, 32768
_K = 64
_LANES = 16
_CHUNKS = _COLS // _LANES
_NBINS = 256
_ROWS_PER_TILE = 2
_U = 8


def _tile_body(x_hbm, out_hbm, row_v, out_v, cidx_v, h0, h1, h2, h3):
    hists = (h0, h1, h2, h3)
    cid = lax.axis_index("c")
    sid = lax.axis_index("s")
    wid = sid * 2 + cid

    lane = lax.iota(jnp.int32, _LANES)
    ones_i = jnp.ones((_LANES,), jnp.int32)
    zeros_i = jnp.zeros((_LANES,), jnp.int32)
    zeros_f = jnp.zeros((_LANES,), jnp.float32)

    def bin_total(d):
        s = hists[0][pl.ds(d * _LANES, _LANES)]
        for h in hists[1:]:
            s = s + h[pl.ds(d * _LANES, _LANES)]
        return jnp.sum(s)

    def scan_bins(d0, kk):
        def cond(st):
            d, acc = st
            return acc + bin_total(d) < kk

        def body(st):
            d, acc = st
            return d - 1, acc + bin_total(d)

        return lax.while_loop(cond, body, (d0, jnp.int32(0)))

    def zero_hist():
        def zh(j, c):
            base = j * (_U * _LANES)
            for t in range(_U):
                for h in hists:
                    h[pl.ds(base + t * _LANES, _LANES)] = zeros_i
            return c

        lax.fori_loop(0, _NBINS // _U, zh, jnp.int32(0))

    def tree_max(ms):
        while len(ms) > 1:
            ms = [jnp.maximum(a, b) for a, b in zip(ms[::2], ms[1::2])]
        return ms[0]

    for rsub in range(_ROWS_PER_TILE):
        row = wid * _ROWS_PER_TILE + rsub
        pltpu.sync_copy(x_hbm.at[row], row_v)

        zero_hist()

        def pA(i, umax):
            base = i * (_U * _LANES)
            ms = []
            for t in range(_U):
                xv = row_v[pl.ds(base + t * _LANES, _LANES)]
                v = jnp.where(xv > 0.0, xv, zeros_f)
                u = plsc.bitcast(v, jnp.int32)
                d = lax.shift_right_logical(u, 24)
                plsc.addupdate_scatter(
                    hists[t % 4], [d * _LANES + lane], ones_i
                )
                ms.append(u)
            return jnp.maximum(umax, tree_max(ms))

        umax = lax.fori_loop(0, _CHUNKS // _U, pA, zeros_i)
        um = jnp.max(umax)
        d_sel, acc = scan_bins(lax.shift_right_logical(um, 24), jnp.int32(_K))
        kk = jnp.int32(_K) - acc

        def pB(i, off):
            base = i * (_U * _LANES)
            for t in range(_U):
                o = base + t * _LANES
                xv = row_v[pl.ds(o, _LANES)]
                v = jnp.where(xv > 0.0, xv, zeros_f)
                u = plsc.bitcast(v, jnp.int32)
                d = lax.shift_right_logical(u, 24)
                gt = d > d_sel
                eq = d == d_sel
                out_v[pl.ds(o, _LANES)] = jnp.where(gt, v, zeros_f)
                eqi = jnp.where(eq, ones_i, zeros_i)
                pos = off + plsc.cumsum(eqi) - eqi
                plsc.store_scatter(cidx_v, [pos], o + lane, mask=eq)
                off = off + plsc.all_reduce_population_count(eq)
            return off

        offv = lax.fori_loop(0, _CHUNKS // _U, pB, zeros_i)
        ncand = jnp.max(offv)

        ncq = (ncand + _LANES - 1) // _LANES
        prefix = d_sel
        for p in range(1, 4):
            shift = 24 - 8 * p
            hs = shift + 8
            zero_hist()

            def pc(ci, umax, shift=shift, hs=hs, prefix=prefix,
                   ncand=ncand):
                cbase = ci * _LANES
                vm = (cbase + lane) < ncand
                cidx = cidx_v[pl.ds(cbase, _LANES)] & (_COLS - 1)
                xg = plsc.load_gather(row_v, [cidx], mask=vm)
                v = jnp.where(xg > 0.0, xg, zeros_f)
                u = plsc.bitcast(v, jnp.int32)
                cand = vm & (lax.shift_right_logical(u, hs) == prefix)
                dg = lax.shift_right_logical(u, shift) & 0xFF
                plsc.addupdate_scatter(
                    hists[0], [dg * _LANES + lane], ones_i, mask=cand
                )
                return jnp.maximum(umax, jnp.where(cand, u, zeros_i))

            umax = lax.fori_loop(0, ncq, pc, zeros_i)
            um = jnp.max(umax)
            d_sel2, acc = scan_bins(
                lax.shift_right_logical(um, shift) & 0xFF, kk
            )
            kk = kk - acc
            prefix = lax.shift_left(prefix, 8) | d_sel2


        def pr(ci, carry, prefix=prefix, kk=kk, ncand=ncand):
            cbase = ci * _LANES
            vm = (cbase + lane) < ncand
            cidx = cidx_v[pl.ds(cbase, _LANES)] & (_COLS - 1)
            xg = plsc.load_gather(row_v, [cidx], mask=vm)
            v = jnp.where(xg > 0.0, xg, zeros_f)
            u = plsc.bitcast(v, jnp.int32)
            gt = vm & (u > prefix)
            eq = vm & (u == prefix)
            cs = plsc.cumsum(jnp.where(eq, ones_i, zeros_i))
            keep = jnp.logical_or(gt, eq & ((cs + carry) <= kk))
            plsc.store_scatter(out_v, [cidx], v, mask=keep)
            return carry + plsc.all_reduce_population_count(eq)

        lax.fori_loop(0, ncq, pr, zeros_i)
        pltpu.sync_copy(out_v, out_hbm.at[row])


@jax.jit
def _topk_sc(x):
    mesh = plsc.VectorSubcoreMesh(core_axis_name="c", subcore_axis_name="s")
    fn = pl.kernel(
        _tile_body,
        out_type=jax.ShapeDtypeStruct((_ROWS, _COLS), jnp.float32),
        mesh=mesh,
        compiler_params=pltpu.CompilerParams(needs_layout_passes=False),
        scratch_types=[
            pltpu.VMEM((_COLS,), jnp.float32),
            pltpu.VMEM((_COLS,), jnp.float32),
            pltpu.VMEM((_COLS,), jnp.int32),
            pltpu.VMEM((_NBINS * _LANES,), jnp.int32),
            pltpu.VMEM((_NBINS * _LANES,), jnp.int32),
            pltpu.VMEM((_NBINS * _LANES,), jnp.int32),
            pltpu.VMEM((_NBINS * _LANES,), jnp.int32),
        ],
    )
    return fn(x)


def kernel(x):
    return _topk_sc(x)


# speedup vs baseline: 1.9259x; 1.9259x over previous
import functools

import jax
import jax.numpy as jnp
from jax import lax
from jax.experimental import pallas as pl
from jax.experimental.pallas import tpu as pltpu
from jax.experimental.pallas import tpu_sc as plsc

_ROWS, _COLS = 64, 32768
_K = 64
_LANES = 16
_CHUNKS = _COLS // _LANES
_NBINS = 256
_ROWS_PER_TILE = 2
_U = 8


def _tile_body(x_hbm, out_hbm, row_v, out_v, cidx_v, h0, h1, h2, h3):
    hists = (h0, h1, h2, h3)
    cid = lax.axis_index("c")
    sid = lax.axis_index("s")
    wid = sid * 2 + cid

    lane = lax.iota(jnp.int32, _LANES)
    ones_i = jnp.ones((_LANES,), jnp.int32)
    zeros_i = jnp.zeros((_LANES,), jnp.int32)
    zeros_f = jnp.zeros((_LANES,), jnp.float32)

    def bin_total(d):
        s = hists[0][pl.ds(d * _LANES, _LANES)]
        for h in hists[1:]:
            s = s + h[pl.ds(d * _LANES, _LANES)]
        return jnp.sum(s)

    def scan_bins(d0, kk):
        def cond(st):
            d, acc = st
            return acc + bin_total(d) < kk

        def body(st):
            d, acc = st
            return d - 1, acc + bin_total(d)

        return lax.while_loop(cond, body, (d0, jnp.int32(0)))

    def zero_hist():
        def zh(j, c):
            base = j * (_U * _LANES)
            for t in range(_U):
                for h in hists:
                    h[pl.ds(base + t * _LANES, _LANES)] = zeros_i
            return c

        lax.fori_loop(0, _NBINS // _U, zh, jnp.int32(0))

    def tree_max(ms):
        while len(ms) > 1:
            ms = [jnp.maximum(a, b) for a, b in zip(ms[::2], ms[1::2])]
        return ms[0]

    for rsub in range(_ROWS_PER_TILE):
        row = wid * _ROWS_PER_TILE + rsub
        pltpu.sync_copy(x_hbm.at[row], row_v)

        zero_hist()

        def pA(i, umax):
            base = i * (_U * _LANES)
            xs = [row_v[pl.ds(base + t * _LANES, _LANES)]
                  for t in range(_U)]
            vs = [jnp.where(x > 0.0, x, zeros_f) for x in xs]
            us = [plsc.bitcast(v, jnp.int32) for v in vs]
            idxs = [lax.shift_right_logical(u, 24) * _LANES + lane
                    for u in us]
            for t in range(_U):
                plsc.addupdate_scatter(hists[t % 4], [idxs[t]], ones_i)
            return jnp.maximum(umax, tree_max(us))

        umax = lax.fori_loop(0, _CHUNKS // _U, pA, zeros_i)
        um = jnp.max(umax)
        d_sel, acc = scan_bins(lax.shift_right_logical(um, 24), jnp.int32(_K))
        kk = jnp.int32(_K) - acc

        def pB(i, off):
            base = i * (_U * _LANES)
            os_ = [base + t * _LANES for t in range(_U)]
            xs = [row_v[pl.ds(o, _LANES)] for o in os_]
            vs = [jnp.where(x > 0.0, x, zeros_f) for x in xs]
            us = [plsc.bitcast(v, jnp.int32) for v in vs]
            ds = [lax.shift_right_logical(u, 24) for u in us]
            gts = [d > d_sel for d in ds]
            eqs = [d == d_sel for d in ds]
            for t in range(_U):
                out_v[pl.ds(os_[t], _LANES)] = jnp.where(
                    gts[t], vs[t], zeros_f
                )
            eqis = [jnp.where(e, ones_i, zeros_i) for e in eqs]
            css = [plsc.cumsum(e) for e in eqis]
            pcs = [plsc.all_reduce_population_count(e) for e in eqs]
            offs = [off]
            for t in range(_U):
                offs.append(offs[-1] + pcs[t])
            for t in range(_U):
                pos = offs[t] + css[t] - eqis[t]
                plsc.store_scatter(
                    cidx_v, [pos], os_[t] + lane, mask=eqs[t]
                )
            return offs[_U]

        offv = lax.fori_loop(0, _CHUNKS // _U, pB, zeros_i)
        ncand = jnp.max(offv)

        ncq = (ncand + _LANES - 1) // _LANES
        prefix = d_sel
        for p in range(1, 4):
            shift = 24 - 8 * p
            hs = shift + 8
            zero_hist()

            def pc(ci, umax, shift=shift, hs=hs, prefix=prefix,
                   ncand=ncand):
                cbase = ci * _LANES
                vm = (cbase + lane) < ncand
                cidx = cidx_v[pl.ds(cbase, _LANES)] & (_COLS - 1)
                xg = plsc.load_gather(row_v, [cidx], mask=vm)
                v = jnp.where(xg > 0.0, xg, zeros_f)
                u = plsc.bitcast(v, jnp.int32)
                cand = vm & (lax.shift_right_logical(u, hs) == prefix)
                dg = lax.shift_right_logical(u, shift) & 0xFF
                plsc.addupdate_scatter(
                    hists[0], [dg * _LANES + lane], ones_i, mask=cand
                )
                return jnp.maximum(umax, jnp.where(cand, u, zeros_i))

            umax = lax.fori_loop(0, ncq, pc, zeros_i)
            um = jnp.max(umax)
            d_sel2, acc = scan_bins(
                lax.shift_right_logical(um, shift) & 0xFF, kk
            )
            kk = kk - acc
            prefix = lax.shift_left(prefix, 8) | d_sel2


        def pr(ci, carry, prefix=prefix, kk=kk, ncand=ncand):
            cbase = ci * _LANES
            vm = (cbase + lane) < ncand
            cidx = cidx_v[pl.ds(cbase, _LANES)] & (_COLS - 1)
            xg = plsc.load_gather(row_v, [cidx], mask=vm)
            v = jnp.where(xg > 0.0, xg, zeros_f)
            u = plsc.bitcast(v, jnp.int32)
            gt = vm & (u > prefix)
            eq = vm & (u == prefix)
            cs = plsc.cumsum(jnp.where(eq, ones_i, zeros_i))
            keep = jnp.logical_or(gt, eq & ((cs + carry) <= kk))
            plsc.store_scatter(out_v, [cidx], v, mask=keep)
            return carry + plsc.all_reduce_population_count(eq)

        lax.fori_loop(0, ncq, pr, zeros_i)
        pltpu.sync_copy(out_v, out_hbm.at[row])


@jax.jit
def _topk_sc(x):
    mesh = plsc.VectorSubcoreMesh(core_axis_name="c", subcore_axis_name="s")
    fn = pl.kernel(
        _tile_body,
        out_type=jax.ShapeDtypeStruct((_ROWS, _COLS), jnp.float32),
        mesh=mesh,
        compiler_params=pltpu.CompilerParams(needs_layout_passes=False),
        scratch_types=[
            pltpu.VMEM((_COLS,), jnp.float32),
            pltpu.VMEM((_COLS,), jnp.float32),
            pltpu.VMEM((_COLS,), jnp.int32),
            pltpu.VMEM((_NBINS * _LANES,), jnp.int32),
            pltpu.VMEM((_NBINS * _LANES,), jnp.int32),
            pltpu.VMEM((_NBINS * _LANES,), jnp.int32),
            pltpu.VMEM((_NBINS * _LANES,), jnp.int32),
        ],
    )
    return fn(x)


def kernel(x):
    return _topk_sc(x)
